# baseline (speedup 1.0000x reference)
import functools

import jax
import jax.numpy as jnp
from jax import lax
from jax.experimental import pallas as pl
from jax.experimental.pallas import tpu as pltpu
from jax.experimental.pallas import tpu_sc as plsc

N = 16384
M = 8192
D = 128
BT = 512
CM = 512
NBLK = N // BT
NCH = M // CM
COMMITMENT = 0.25


WINDOWS = ((0, 4096), (4096, 8192))


def _argmin_body(x_ref, sx_ref, w_ref, sw_ref, idx_ref, dsum_ref):
    x = x_ref[...]
    sx = sx_ref[...]
    run_v = jnp.full((BT, 1), jnp.inf, jnp.float32)
    run_i = jnp.zeros((BT, 1), jnp.int32)
    for a, b in WINDOWS:
        L = b - a
        wc = w_ref[pl.ds(a, L), :]
        swc = sw_ref[:, pl.ds(a, L)]
        mm = lax.dot_general(x, wc, (((1,), (1,)), ((), ())),
                             preferred_element_type=jnp.float32)
        d = (sx + swc) + mm
        wv = jnp.min(d, axis=1, keepdims=True)
        ii = lax.broadcasted_iota(jnp.int32, (BT, L), 1)
        wi = jnp.min(jnp.where(d == wv, ii, M), axis=1, keepdims=True) + a
        pick = (wv < run_v) | ((wv == run_v) & (wi < run_i))
        run_v = jnp.where(pick, wv, run_v)
        run_i = jnp.where(pick, wi, run_i)
        run_v = run_v.astype(jnp.bfloat16).astype(jnp.float32)
    idx_ref[...] = run_i
    dsum_ref[0] = jnp.sum(run_v, axis=0, keepdims=True)


def _distance_argmin(x, sx, w, sw):
    return pl.pallas_call(
        _argmin_body,
        grid=(NBLK,),
        in_specs=[
            pl.BlockSpec((BT, D), lambda i: (i, 0)),
            pl.BlockSpec((BT, 1), lambda i: (i, 0)),
            pl.BlockSpec((M, D), lambda i: (0, 0)),
            pl.BlockSpec((1, M), lambda i: (0, 0)),
        ],
        out_specs=[
            pl.BlockSpec((BT, 1), lambda i: (i, 0)),
            pl.BlockSpec((1, 1, 1), lambda i: (i, 0, 0)),
        ],
        out_shape=[
            jax.ShapeDtypeStruct((N, 1), jnp.int32),
            jax.ShapeDtypeStruct((NBLK, 1, 1), jnp.float32),
        ],
    )(x, sx, w, sw)


_NC, _NS = 2, 16
NW = _NC * _NS
BPW = N // NW
CH = 128
NCHG = BPW // CH


@functools.cache
def _make_sc_gather():
    mesh = plsc.VectorSubcoreMesh(core_axis_name="c", subcore_axis_name="s")

    @functools.partial(
        pl.kernel,
        mesh=mesh,
        out_type=jax.ShapeDtypeStruct((N, D), jnp.float32),
        scratch_types=[
            pltpu.VMEM((NCHG, CH), jnp.int32),
            pltpu.VMEM((BPW, D), jnp.float32),
            pltpu.SemaphoreType.DMA,
        ],
    )
    def _sc_gather(table_hbm, idx_hbm, out_hbm, idx_v, rows_v, sem):
        wid = lax.axis_index("s") * _NC + lax.axis_index("c")
        base = wid * BPW
        pltpu.sync_copy(idx_hbm.at[pl.ds(wid * NCHG, NCHG)], idx_v)
        copies = [
            pltpu.async_copy(table_hbm.at[idx_v.at[j]],
                             rows_v.at[pl.ds(j * CH, CH)], sem)
            for j in range(NCHG)
        ]
        for c in copies:
            c.wait()
        pltpu.sync_copy(rows_v, out_hbm.at[pl.ds(base, BPW)])

    return _sc_gather


def kernel(inputs, weight):
    xb = inputs.astype(jnp.bfloat16)
    wb = weight.astype(jnp.bfloat16)
    wq = wb.astype(jnp.float32)
    sx = jnp.sum(inputs ** 2, axis=1, keepdims=True)
    sw = jnp.sum(weight ** 2, axis=1)[None, :]
    idx2d, dsums = _distance_argmin(xb, sx, wb * jnp.bfloat16(-2.0), sw)
    encoding_indices = idx2d.reshape(N)
    quantized = _make_sc_gather()(wq, idx2d.reshape(N // CH, CH))
    m = jnp.sum(dsums) / (N * D)
    loss = m + COMMITMENT * m
    quantized_st = inputs + (quantized - inputs)
    quantized_2d = quantized_st[:, :, None, None]
    return (quantized_2d, quantized_st, loss, encoding_indices)


# speedup vs baseline: 1.0532x; 1.0532x over previous
import functools

import jax
import jax.numpy as jnp
from jax import lax
from jax.experimental import pallas as pl
from jax.experimental.pallas import tpu as pltpu
from jax.experimental.pallas import tpu_sc as plsc

N = 16384
M = 8192
D = 128
BT = 512
CM = 512
NBLK = N // BT
NCH = M // CM
COMMITMENT = 0.25


WINDOWS = ((0, 4096), (4096, 8192))


def _argmin_body(x_ref, sx_ref, w_ref, sw_ref, idx_ref, dsum_ref):
    x = x_ref[...]
    sx = sx_ref[...]
    run_v = jnp.full((BT, 1), jnp.inf, jnp.float32)
    run_i = jnp.zeros((BT, 1), jnp.int32)
    for a, b in WINDOWS:
        L = b - a
        wc = w_ref[pl.ds(a, L), :]
        swc = sw_ref[:, pl.ds(a, L)]
        mm = lax.dot_general(x, wc, (((1,), (1,)), ((), ())),
                             preferred_element_type=jnp.float32)
        d = (sx + swc) - 2.0 * mm
        wv = jnp.min(d, axis=1, keepdims=True)
        ii = lax.broadcasted_iota(jnp.int32, (BT, L), 1)
        wi = jnp.min(jnp.where(d == wv, ii, M), axis=1, keepdims=True) + a
        pick = (wv < run_v) | ((wv == run_v) & (wi < run_i))
        run_v = jnp.where(pick, wv, run_v)
        run_i = jnp.where(pick, wi, run_i)
        run_v = run_v.astype(jnp.bfloat16).astype(jnp.float32)
    idx_ref[...] = run_i
    dsum_ref[0] = jnp.sum(run_v, axis=0, keepdims=True)


def _distance_argmin(x, sx, w, sw):
    return pl.pallas_call(
        _argmin_body,
        grid=(NBLK,),
        in_specs=[
            pl.BlockSpec((BT, D), lambda i: (i, 0)),
            pl.BlockSpec((BT, 1), lambda i: (i, 0)),
            pl.BlockSpec((M, D), lambda i: (0, 0)),
            pl.BlockSpec((1, M), lambda i: (0, 0)),
        ],
        out_specs=[
            pl.BlockSpec((BT, 1), lambda i: (i, 0)),
            pl.BlockSpec((1, 1, 1), lambda i: (i, 0, 0)),
        ],
        out_shape=[
            jax.ShapeDtypeStruct((N, 1), jnp.int32),
            jax.ShapeDtypeStruct((NBLK, 1, 1), jnp.float32),
        ],
    )(x, sx, w, sw)


_NC, _NS = 2, 16
NW = _NC * _NS
BPW = N // NW
CH = 128
NCHG = BPW // CH


@functools.cache
def _make_sc_gather():
    mesh = plsc.VectorSubcoreMesh(core_axis_name="c", subcore_axis_name="s")

    @functools.partial(
        pl.kernel,
        mesh=mesh,
        out_type=jax.ShapeDtypeStruct((N, D), jnp.float32),
        scratch_types=[
            pltpu.VMEM((NCHG, CH), jnp.int32),
            pltpu.VMEM((BPW, D), jnp.float32),
            pltpu.SemaphoreType.DMA,
        ],
    )
    def _sc_gather(table_hbm, idx_hbm, out_hbm, idx_v, rows_v, sem):
        wid = lax.axis_index("s") * _NC + lax.axis_index("c")
        base = wid * BPW
        pltpu.sync_copy(idx_hbm.at[pl.ds(wid * NCHG, NCHG)], idx_v)
        copies = [
            pltpu.async_copy(table_hbm.at[idx_v.at[j]],
                             rows_v.at[pl.ds(j * CH, CH)], sem)
            for j in range(NCHG)
        ]
        for c in copies:
            c.wait()
        pltpu.sync_copy(rows_v, out_hbm.at[pl.ds(base, BPW)])

    return _sc_gather


def kernel(inputs, weight):
    xb = inputs.astype(jnp.bfloat16)
    wb = weight.astype(jnp.bfloat16)
    wq = wb.astype(jnp.float32)
    sx = jnp.sum(inputs ** 2, axis=1, keepdims=True)
    sw = jnp.sum(weight ** 2, axis=1)[None, :]
    idx2d, dsums = _distance_argmin(xb, sx, wb, sw)
    encoding_indices = idx2d.reshape(N)
    quantized = _make_sc_gather()(wq, idx2d.reshape(N // CH, CH))
    m = jnp.sum(dsums) / (N * D)
    loss = m + COMMITMENT * m
    quantized_st = inputs + (quantized - inputs)
    quantized_2d = quantized_st[:, :, None, None]
    return (quantized_2d, quantized_st, loss, encoding_indices)


# speedup vs baseline: 1.0647x; 1.0109x over previous
import functools

import jax
import jax.numpy as jnp
from jax import lax
from jax.experimental import pallas as pl
from jax.experimental.pallas import tpu as pltpu
from jax.experimental.pallas import tpu_sc as plsc

N = 16384
M = 8192
D = 128
BT = 1024
CM = 512
NBLK = N // BT
NCH = M // CM
COMMITMENT = 0.25


WINDOWS = ((0, 4096), (4096, 8192))


def _argmin_body(x_ref, sx_ref, w_ref, sw_ref, idx_ref, dsum_ref):
    x = x_ref[...]
    sx = sx_ref[...]
    run_v = jnp.full((BT, 1), jnp.inf, jnp.float32)
    run_i = jnp.zeros((BT, 1), jnp.int32)
    for a, b in WINDOWS:
        L = b - a
        wc = w_ref[pl.ds(a, L), :]
        swc = sw_ref[:, pl.ds(a, L)]
        mm = lax.dot_general(x, wc, (((1,), (1,)), ((), ())),
                             preferred_element_type=jnp.float32)
        d = (sx + swc) - 2.0 * mm
        wv = jnp.min(d, axis=1, keepdims=True)
        ii = lax.broadcasted_iota(jnp.int32, (BT, L), 1)
        wi = jnp.min(jnp.where(d == wv, ii, M), axis=1, keepdims=True) + a
        pick = (wv < run_v) | ((wv == run_v) & (wi < run_i))
        run_v = jnp.where(pick, wv, run_v)
        run_i = jnp.where(pick, wi, run_i)
        run_v = run_v.astype(jnp.bfloat16).astype(jnp.float32)
    idx_ref[...] = run_i
    dsum_ref[0] = jnp.sum(run_v, axis=0, keepdims=True)


def _distance_argmin(x, sx, w, sw):
    return pl.pallas_call(
        _argmin_body,
        grid=(NBLK,),
        in_specs=[
            pl.BlockSpec((BT, D), lambda i: (i, 0)),
            pl.BlockSpec((BT, 1), lambda i: (i, 0)),
            pl.BlockSpec((M, D), lambda i: (0, 0)),
            pl.BlockSpec((1, M), lambda i: (0, 0)),
        ],
        out_specs=[
            pl.BlockSpec((BT, 1), lambda i: (i, 0)),
            pl.BlockSpec((1, 1, 1), lambda i: (i, 0, 0)),
        ],
        out_shape=[
            jax.ShapeDtypeStruct((N, 1), jnp.int32),
            jax.ShapeDtypeStruct((NBLK, 1, 1), jnp.float32),
        ],
    )(x, sx, w, sw)


_NC, _NS = 2, 16
NW = _NC * _NS
BPW = N // NW
CH = 128
NCHG = BPW // CH


@functools.cache
def _make_sc_gather():
    mesh = plsc.VectorSubcoreMesh(core_axis_name="c", subcore_axis_name="s")

    @functools.partial(
        pl.kernel,
        mesh=mesh,
        out_type=jax.ShapeDtypeStruct((N, D), jnp.float32),
        scratch_types=[
            pltpu.VMEM((NCHG, CH), jnp.int32),
            pltpu.VMEM((BPW, D), jnp.float32),
            pltpu.SemaphoreType.DMA,
        ],
    )
    def _sc_gather(table_hbm, idx_hbm, out_hbm, idx_v, rows_v, sem):
        wid = lax.axis_index("s") * _NC + lax.axis_index("c")
        base = wid * BPW
        pltpu.sync_copy(idx_hbm.at[pl.ds(wid * NCHG, NCHG)], idx_v)
        copies = [
            pltpu.async_copy(table_hbm.at[idx_v.at[j]],
                             rows_v.at[pl.ds(j * CH, CH)], sem)
            for j in range(NCHG)
        ]
        for c in copies:
            c.wait()
        pltpu.sync_copy(rows_v, out_hbm.at[pl.ds(base, BPW)])

    return _sc_gather


def kernel(inputs, weight):
    xb = inputs.astype(jnp.bfloat16)
    wb = weight.astype(jnp.bfloat16)
    wq = wb.astype(jnp.float32)
    sx = jnp.sum(inputs ** 2, axis=1, keepdims=True)
    sw = jnp.sum(weight ** 2, axis=1)[None, :]
    idx2d, dsums = _distance_argmin(xb, sx, wb, sw)
    encoding_indices = idx2d.reshape(N)
    quantized = _make_sc_gather()(wq, idx2d.reshape(N // CH, CH))
    m = jnp.sum(dsums) / (N * D)
    loss = m + COMMITMENT * m
    quantized_st = inputs + (quantized - inputs)
    quantized_2d = quantized_st[:, :, None, None]
    return (quantized_2d, quantized_st, loss, encoding_indices)


# speedup vs baseline: 1.0649x; 1.0002x over previous
import functools

import jax
import jax.numpy as jnp
from jax import lax
from jax.experimental import pallas as pl
from jax.experimental.pallas import tpu as pltpu
from jax.experimental.pallas import tpu_sc as plsc

N = 16384
M = 8192
D = 128
BT = 1024
NBLK = N // BT
COMMITMENT = 0.25


WINDOWS = ((0, 4096), (4096, 8192))


def _argmin_body(x_ref, sx_ref, w_ref, sw_ref, idx_ref, dsum_ref):
    x = x_ref[...]
    sx = sx_ref[...]
    run_v = jnp.full((BT, 1), jnp.inf, jnp.float32)
    run_i = jnp.zeros((BT, 1), jnp.int32)
    for a, b in WINDOWS:
        L = b - a
        wc = w_ref[pl.ds(a, L), :]
        swc = sw_ref[:, pl.ds(a, L)]
        mm = lax.dot_general(x, wc, (((1,), (1,)), ((), ())),
                             preferred_element_type=jnp.float32)
        d = (sx + swc) - 2.0 * mm
        wv = jnp.min(d, axis=1, keepdims=True)
        ii = lax.broadcasted_iota(jnp.int32, (BT, L), 1)
        wi = jnp.min(jnp.where(d == wv, ii, M), axis=1, keepdims=True) + a
        pick = (wv < run_v) | ((wv == run_v) & (wi < run_i))
        run_v = jnp.where(pick, wv, run_v)
        run_i = jnp.where(pick, wi, run_i)
        run_v = run_v.astype(jnp.bfloat16).astype(jnp.float32)
    idx_ref[...] = run_i
    dsum_ref[0] = jnp.sum(run_v, axis=0, keepdims=True)


def _distance_argmin(x, sx, w, sw):
    return pl.pallas_call(
        _argmin_body,
        grid=(NBLK,),
        in_specs=[
            pl.BlockSpec((BT, D), lambda i: (i, 0)),
            pl.BlockSpec((BT, 1), lambda i: (i, 0)),
            pl.BlockSpec((M, D), lambda i: (0, 0)),
            pl.BlockSpec((1, M), lambda i: (0, 0)),
        ],
        out_specs=[
            pl.BlockSpec((BT, 1), lambda i: (i, 0)),
            pl.BlockSpec((1, 1, 1), lambda i: (i, 0, 0)),
        ],
        out_shape=[
            jax.ShapeDtypeStruct((N, 1), jnp.int32),
            jax.ShapeDtypeStruct((NBLK, 1, 1), jnp.float32),
        ],
    )(x, sx, w, sw)


_NC, _NS = 2, 16
NW = _NC * _NS
BPW = N // NW
CH = 128
NCHG = BPW // CH


@functools.cache
def _make_sc_gather():
    mesh = plsc.VectorSubcoreMesh(core_axis_name="c", subcore_axis_name="s")

    @functools.partial(
        pl.kernel,
        mesh=mesh,
        out_type=jax.ShapeDtypeStruct((N, D), jnp.float32),
        scratch_types=[
            pltpu.VMEM((NCHG, CH), jnp.int32),
            pltpu.VMEM((BPW, D), jnp.float32),
            pltpu.SemaphoreType.DMA,
        ],
    )
    def _sc_gather(table_hbm, idx_hbm, out_hbm, idx_v, rows_v, sem):
        wid = lax.axis_index("s") * _NC + lax.axis_index("c")
        base = wid * BPW
        pltpu.sync_copy(idx_hbm.at[pl.ds(wid * NCHG, NCHG)], idx_v)
        copies = [
            pltpu.async_copy(table_hbm.at[idx_v.at[j]],
                             rows_v.at[pl.ds(j * CH, CH)], sem)
            for j in range(NCHG)
        ]
        for c in copies:
            c.wait()
        pltpu.sync_copy(rows_v, out_hbm.at[pl.ds(base, BPW)])

    return _sc_gather


def kernel(inputs, weight):
    xb = inputs.astype(jnp.bfloat16)
    wb = weight.astype(jnp.bfloat16)
    wq = wb.astype(jnp.float32)
    sx = jnp.sum(inputs ** 2, axis=1, keepdims=True)
    sw = jnp.sum(weight ** 2, axis=1)[None, :]
    idx2d, dsums = _distance_argmin(xb, sx, wb, sw)
    encoding_indices = idx2d.reshape(N)
    quantized = _make_sc_gather()(wq, idx2d.reshape(N // CH, CH))
    m = jnp.sum(dsums) / (N * D)
    loss = m + COMMITMENT * m
    quantized_st = inputs + (quantized - inputs)
    quantized_2d = quantized_st[:, :, None, None]
    return (quantized_2d, quantized_st, loss, encoding_indices)
